# speedup vs baseline: 1.3583x; 1.3583x over previous
import functools

import jax
import jax.numpy as jnp
from jax import lax
from jax.experimental import pallas as pl
from jax.experimental.pallas import tpu as pltpu
from jax.experimental.pallas import tpu_sc as plsc

B, S = 4, 2048
E = 8
D = 1024
EXP_D = 128
C = 8
NC, NS, L = 2, 16, 16
NW = NC * NS

A_CHUNKS = 8
A_IDX = S // A_CHUNKS
A_ROWS = 64
A_GATHERS = A_IDX // A_ROWS

B_ROWS = 256
B_GATHERS = S // B_ROWS


def _sc_body(vocab, x_hbm, emb_hbm, eemb_hbm, pa_hbm, pb_hbm,
             idxa_v, bufa_v, acca_v, idxb_v, bufb_v, accb_v, sem):
    wid = lax.axis_index("s") * NC + lax.axis_index("c")

    b = wid // A_CHUNKS
    c = wid % A_CHUNKS
    base = b * S + c * A_IDX
    pltpu.sync_copy(x_hbm.at[pl.ds(base, A_IDX)], idxa_v)
    zero16 = jnp.zeros((L,), jnp.float32)
    for j in range(D // L):
        acca_v[pl.ds(j * L, L)] = zero16

    def a_chunk(ci, carry):
        pltpu.async_copy(
            emb_hbm.at[idxa_v.at[pl.ds(ci * A_ROWS, A_ROWS)]], bufa_v, sem
        ).wait()

        def a_row(r, carry2):
            for j in range(D // L):
                sl = pl.ds(j * L, L)
                plsc.addupdate(acca_v.at[sl], bufa_v[r, sl])
            return carry2

        return lax.fori_loop(0, A_ROWS, a_row, carry)

    lax.fori_loop(0, A_GATHERS, a_chunk, 0)
    pltpu.sync_copy(acca_v, pa_hbm.at[wid])

    e = wid // B
    bb = wid % B
    pltpu.sync_copy(x_hbm.at[pl.ds(bb * S, S)], idxb_v)
    off = e * vocab
    for j in range(S // L):
        sl = pl.ds(j * L, L)
        idxb_v[sl] = idxb_v[sl] + off
    for j in range(EXP_D // L):
        accb_v[pl.ds(j * L, L)] = zero16

    def b_chunk(ci, carry):
        pltpu.async_copy(
            eemb_hbm.at[idxb_v.at[pl.ds(ci * B_ROWS, B_ROWS)]], bufb_v, sem
        ).wait()

        def b_row(r, carry2):
            for j in range(EXP_D // L):
                sl = pl.ds(j * L, L)
                plsc.addupdate(accb_v.at[sl], bufb_v[r, sl])
            return carry2

        return lax.fori_loop(0, B_ROWS, b_row, carry)

    lax.fori_loop(0, B_GATHERS, b_chunk, 0)
    pltpu.sync_copy(accb_v, pb_hbm.at[wid])


def _sc_pools(x_flat, emb, eemb_flat):
    vocab = emb.shape[0]
    mesh = plsc.VectorSubcoreMesh(core_axis_name="c", subcore_axis_name="s")
    body = functools.partial(_sc_body, vocab)
    return pl.kernel(
        body,
        out_type=(
            jax.ShapeDtypeStruct((NW, D), jnp.float32),
            jax.ShapeDtypeStruct((NW, EXP_D), jnp.float32),
        ),
        mesh=mesh,
        scratch_types=[
            pltpu.VMEM((A_IDX,), jnp.int32),
            pltpu.VMEM((A_ROWS, D), jnp.float32),
            pltpu.VMEM((D,), jnp.float32),
            pltpu.VMEM((S,), jnp.int32),
            pltpu.VMEM((B_ROWS, EXP_D), jnp.float32),
            pltpu.VMEM((EXP_D,), jnp.float32),
            pltpu.SemaphoreType.DMA,
        ],
    )(x_flat, emb, eemb_flat)


def _tc_body(pa_ref, pb_ref, w1_ref, b1_ref, w2_ref, b2_ref,
             expw_ref, expb_ref, out_ref):
    inv_s = 1.0 / S
    pooled = pa_ref[...].reshape(B, A_CHUNKS, D).sum(axis=1) * inv_s
    h = jnp.maximum(pooled @ w1_ref[...] + b1_ref[...], 0.0)
    gates = h @ w2_ref[...] + b2_ref[...]
    m = jnp.max(gates, axis=-1, keepdims=True)
    pexp = jnp.exp(gates - m)
    probs = pexp / jnp.sum(pexp, axis=-1, keepdims=True)
    idx = lax.broadcasted_iota(jnp.int32, (B, E), 1)
    top1 = jnp.max(probs, axis=-1, keepdims=True)
    e1 = jnp.min(jnp.where(probs >= top1, idx, E), axis=-1, keepdims=True)
    m1 = idx == e1
    probs2 = jnp.where(m1, -jnp.inf, probs)
    top2 = jnp.max(probs2, axis=-1, keepdims=True)
    e2 = jnp.min(jnp.where(probs2 >= top2, idx, E), axis=-1, keepdims=True)
    m2 = idx == e2
    denom = top1 + top2
    coeff = (jnp.where(m1, top1, 0.0) + jnp.where(m2, top2, 0.0)) / denom
    ep = pb_ref[...].reshape(E, B, EXP_D) * inv_s
    acc = jnp.zeros((B, C), jnp.float32)
    for e in range(E):
        y = ep[e] @ expw_ref[e] + expb_ref[e]
        acc = acc + coeff[:, e:e + 1] * y
    out_ref[...] = acc


def _tc_combine(pa, pb, gate_W1, gate_b1, gate_W2, gate_b2, exp_W, exp_b):
    return pl.pallas_call(
        _tc_body,
        out_shape=jax.ShapeDtypeStruct((B, C), jnp.float32),
    )(pa, pb, gate_W1, gate_b1, gate_W2, gate_b2, exp_W, exp_b)


def kernel(x, emb, gate_W1, gate_b1, gate_W2, gate_b2, exp_emb, exp_W, exp_b):
    vocab = emb.shape[0]
    x_flat = x.reshape(-1).astype(jnp.int32)
    eemb_flat = exp_emb.reshape(E * vocab, EXP_D)
    pa, pb = _sc_pools(x_flat, emb, eemb_flat)
    return _tc_combine(pa, pb, gate_W1, gate_b1, gate_W2, gate_b2,
                       exp_W, exp_b)


# speedup vs baseline: 3.0955x; 2.2789x over previous
import functools

import jax
import jax.numpy as jnp
from jax import lax
from jax.experimental import pallas as pl
from jax.experimental.pallas import tpu as pltpu
from jax.experimental.pallas import tpu_sc as plsc

B, S = 4, 2048
E = 8
D = 1024
EXP_D = 128
C = 8
NC, NS, L = 2, 16, 16
NW = NC * NS

A_CHUNKS = 8
A_IDX = S // A_CHUNKS
A_ROWS = 64
A_GATHERS = A_IDX // A_ROWS

B_ROWS = 256
B_GATHERS = S // B_ROWS


def _accum_rows(buf_v, acc_v, n_rows, n_slices, half_slices, row_unroll):
    zero16 = jnp.zeros((L,), jnp.float32)
    for h0 in range(0, n_slices, half_slices):
        hs = min(half_slices, n_slices - h0)

        def row_body(i, accs, h0=h0, hs=hs):
            out = list(accs)
            for u in range(row_unroll):
                r = i * row_unroll + u
                for j in range(hs):
                    sl = pl.ds((h0 + j) * L, L)
                    out[j] = out[j] + buf_v[r, sl]
            return tuple(out)

        accs = lax.fori_loop(0, n_rows // row_unroll, row_body,
                             (zero16,) * hs)
        for j in range(hs):
            plsc.addupdate(acc_v.at[pl.ds((h0 + j) * L, L)], accs[j])


def _sc_body(vocab, x_hbm, emb_hbm, eemb_hbm, pa_hbm, pb_hbm,
             idxa_v, bufa_v, acca_v, idxb_v, bufb_v, accb_v, sem):
    wid = lax.axis_index("s") * NC + lax.axis_index("c")
    zero16 = jnp.zeros((L,), jnp.float32)

    b = wid // A_CHUNKS
    c = wid % A_CHUNKS
    base = b * S + c * A_IDX
    pltpu.sync_copy(x_hbm.at[pl.ds(base, A_IDX)], idxa_v)
    for j in range(D // L):
        acca_v[pl.ds(j * L, L)] = zero16

    def a_chunk(ci, carry):
        pltpu.async_copy(
            emb_hbm.at[idxa_v.at[pl.ds(ci * A_ROWS, A_ROWS)]], bufa_v, sem
        ).wait()
        _accum_rows(bufa_v, acca_v, A_ROWS, D // L,
                    half_slices=32, row_unroll=1)
        return carry

    lax.fori_loop(0, A_GATHERS, a_chunk, 0)
    pltpu.sync_copy(acca_v, pa_hbm.at[wid])

    e = wid // B
    bb = wid % B
    pltpu.sync_copy(x_hbm.at[pl.ds(bb * S, S)], idxb_v)
    off = e * vocab
    for j in range(S // L):
        sl = pl.ds(j * L, L)
        idxb_v[sl] = idxb_v[sl] + off
    for j in range(EXP_D // L):
        accb_v[pl.ds(j * L, L)] = zero16

    def b_chunk(ci, carry):
        pltpu.async_copy(
            eemb_hbm.at[idxb_v.at[pl.ds(ci * B_ROWS, B_ROWS)]], bufb_v, sem
        ).wait()
        _accum_rows(bufb_v, accb_v, B_ROWS, EXP_D // L,
                    half_slices=8, row_unroll=4)
        return carry

    lax.fori_loop(0, B_GATHERS, b_chunk, 0)
    pltpu.sync_copy(accb_v, pb_hbm.at[wid])


def _sc_pools(x_flat, emb, eemb_flat):
    vocab = emb.shape[0]
    mesh = plsc.VectorSubcoreMesh(core_axis_name="c", subcore_axis_name="s")
    body = functools.partial(_sc_body, vocab)
    return pl.kernel(
        body,
        out_type=(
            jax.ShapeDtypeStruct((NW, D), jnp.float32),
            jax.ShapeDtypeStruct((NW, EXP_D), jnp.float32),
        ),
        mesh=mesh,
        scratch_types=[
            pltpu.VMEM((A_IDX,), jnp.int32),
            pltpu.VMEM((A_ROWS, D), jnp.float32),
            pltpu.VMEM((D,), jnp.float32),
            pltpu.VMEM((S,), jnp.int32),
            pltpu.VMEM((B_ROWS, EXP_D), jnp.float32),
            pltpu.VMEM((EXP_D,), jnp.float32),
            pltpu.SemaphoreType.DMA,
        ],
    )(x_flat, emb, eemb_flat)


def _tc_body(pa_ref, pb_ref, w1_ref, b1_ref, w2_ref, b2_ref,
             expw_ref, expb_ref, out_ref):
    inv_s = 1.0 / S
    pooled = pa_ref[...].reshape(B, A_CHUNKS, D).sum(axis=1) * inv_s
    h = jnp.maximum(pooled @ w1_ref[...] + b1_ref[...], 0.0)
    gates = h @ w2_ref[...] + b2_ref[...]
    m = jnp.max(gates, axis=-1, keepdims=True)
    pexp = jnp.exp(gates - m)
    probs = pexp / jnp.sum(pexp, axis=-1, keepdims=True)
    idx = lax.broadcasted_iota(jnp.int32, (B, E), 1)
    top1 = jnp.max(probs, axis=-1, keepdims=True)
    e1 = jnp.min(jnp.where(probs >= top1, idx, E), axis=-1, keepdims=True)
    m1 = idx == e1
    probs2 = jnp.where(m1, -jnp.inf, probs)
    top2 = jnp.max(probs2, axis=-1, keepdims=True)
    e2 = jnp.min(jnp.where(probs2 >= top2, idx, E), axis=-1, keepdims=True)
    m2 = idx == e2
    denom = top1 + top2
    coeff = (jnp.where(m1, top1, 0.0) + jnp.where(m2, top2, 0.0)) / denom
    ep = pb_ref[...].reshape(E, B, EXP_D) * inv_s
    acc = jnp.zeros((B, C), jnp.float32)
    for e in range(E):
        y = ep[e] @ expw_ref[e] + expb_ref[e]
        acc = acc + coeff[:, e:e + 1] * y
    out_ref[...] = acc


def _tc_combine(pa, pb, gate_W1, gate_b1, gate_W2, gate_b2, exp_W, exp_b):
    return pl.pallas_call(
        _tc_body,
        out_shape=jax.ShapeDtypeStruct((B, C), jnp.float32),
    )(pa, pb, gate_W1, gate_b1, gate_W2, gate_b2, exp_W, exp_b)


def kernel(x, emb, gate_W1, gate_b1, gate_W2, gate_b2, exp_emb, exp_W, exp_b):
    vocab = emb.shape[0]
    x_flat = x.reshape(-1).astype(jnp.int32)
    eemb_flat = exp_emb.reshape(E * vocab, EXP_D)
    pa, pb = _sc_pools(x_flat, emb, eemb_flat)
    return _tc_combine(pa, pb, gate_W1, gate_b1, gate_W2, gate_b2,
                       exp_W, exp_b)


# speedup vs baseline: 3.5656x; 1.1519x over previous
import functools

import jax
import jax.numpy as jnp
from jax import lax
from jax.experimental import pallas as pl
from jax.experimental.pallas import tpu as pltpu
from jax.experimental.pallas import tpu_sc as plsc

B, S = 4, 2048
E = 8
D = 1024
EXP_D = 128
C = 8
NC, NS, L = 2, 16, 16
NW = NC * NS

A_CHUNKS = 8
A_IDX = S // A_CHUNKS
A_ROWS = 32
A_GATHERS = A_IDX // A_ROWS

B_ROWS = 128
B_GATHERS = S // B_ROWS


def _accum_rows(buf_v, acc_v, n_rows, n_slices, half_slices, row_unroll):
    zero16 = jnp.zeros((L,), jnp.float32)
    for h0 in range(0, n_slices, half_slices):
        hs = min(half_slices, n_slices - h0)

        def row_body(i, accs, h0=h0, hs=hs):
            out = list(accs)
            for u in range(row_unroll):
                r = i * row_unroll + u
                for j in range(hs):
                    sl = pl.ds((h0 + j) * L, L)
                    out[j] = out[j] + buf_v[r, sl]
            return tuple(out)

        accs = lax.fori_loop(0, n_rows // row_unroll, row_body,
                             (zero16,) * hs)
        for j in range(hs):
            plsc.addupdate(acc_v.at[pl.ds((h0 + j) * L, L)], accs[j])


def _sc_body(vocab, x_hbm, emb_hbm, eemb_hbm, pa_hbm, pb_hbm,
             idxa_v, bufa0_v, bufa1_v, acca_v,
             idxb_v, bufb0_v, bufb1_v, accb_v, sem0, sem1):
    wid = lax.axis_index("s") * NC + lax.axis_index("c")
    zero16 = jnp.zeros((L,), jnp.float32)

    b = wid // A_CHUNKS
    c = wid % A_CHUNKS
    base = b * S + c * A_IDX
    pltpu.sync_copy(x_hbm.at[pl.ds(base, A_IDX)], idxa_v)

    bufs_a = (bufa0_v, bufa1_v)
    sems = (sem0, sem1)

    def a_start(ci):
        return pltpu.async_copy(
            emb_hbm.at[idxa_v.at[pl.ds(ci * A_ROWS, A_ROWS)]],
            bufs_a[ci % 2], sems[ci % 2])

    d_prev = a_start(0)

    e = wid // B
    bb = wid % B
    pltpu.sync_copy(x_hbm.at[pl.ds(bb * S, S)], idxb_v)
    off = e * vocab
    for j in range(S // L):
        sl = pl.ds(j * L, L)
        idxb_v[sl] = idxb_v[sl] + off
    for j in range(D // L):
        acca_v[pl.ds(j * L, L)] = zero16
    for j in range(EXP_D // L):
        accb_v[pl.ds(j * L, L)] = zero16

    for ci in range(A_GATHERS):
        d_next = a_start(ci + 1) if ci + 1 < A_GATHERS else None
        d_prev.wait()
        _accum_rows(bufs_a[ci % 2], acca_v, A_ROWS, D // L,
                    half_slices=32, row_unroll=1)
        d_prev = d_next
    pltpu.sync_copy(acca_v, pa_hbm.at[wid])

    bufs_b = (bufb0_v, bufb1_v)

    def b_start(ci):
        return pltpu.async_copy(
            eemb_hbm.at[idxb_v.at[pl.ds(ci * B_ROWS, B_ROWS)]],
            bufs_b[ci % 2], sems[ci % 2])

    d_prev = b_start(0)
    for ci in range(B_GATHERS):
        d_next = b_start(ci + 1) if ci + 1 < B_GATHERS else None
        d_prev.wait()
        _accum_rows(bufs_b[ci % 2], accb_v, B_ROWS, EXP_D // L,
                    half_slices=8, row_unroll=4)
        d_prev = d_next
    pltpu.sync_copy(accb_v, pb_hbm.at[wid])


def _sc_pools(x_flat, emb, eemb_flat):
    vocab = emb.shape[0]
    mesh = plsc.VectorSubcoreMesh(core_axis_name="c", subcore_axis_name="s")
    body = functools.partial(_sc_body, vocab)
    return pl.kernel(
        body,
        out_type=(
            jax.ShapeDtypeStruct((NW, D), jnp.float32),
            jax.ShapeDtypeStruct((NW, EXP_D), jnp.float32),
        ),
        mesh=mesh,
        scratch_types=[
            pltpu.VMEM((A_IDX,), jnp.int32),
            pltpu.VMEM((A_ROWS, D), jnp.float32),
            pltpu.VMEM((A_ROWS, D), jnp.float32),
            pltpu.VMEM((D,), jnp.float32),
            pltpu.VMEM((S,), jnp.int32),
            pltpu.VMEM((B_ROWS, EXP_D), jnp.float32),
            pltpu.VMEM((B_ROWS, EXP_D), jnp.float32),
            pltpu.VMEM((EXP_D,), jnp.float32),
            pltpu.SemaphoreType.DMA,
            pltpu.SemaphoreType.DMA,
        ],
    )(x_flat, emb, eemb_flat)


def _tc_body(pa_ref, pb_ref, w1_ref, b1_ref, w2_ref, b2_ref,
             expw_ref, expb_ref, out_ref):
    inv_s = 1.0 / S
    pooled = pa_ref[...].reshape(B, A_CHUNKS, D).sum(axis=1) * inv_s
    h = jnp.maximum(pooled @ w1_ref[...] + b1_ref[...], 0.0)
    gates = h @ w2_ref[...] + b2_ref[...]
    m = jnp.max(gates, axis=-1, keepdims=True)
    pexp = jnp.exp(gates - m)
    probs = pexp / jnp.sum(pexp, axis=-1, keepdims=True)
    idx = lax.broadcasted_iota(jnp.int32, (B, E), 1)
    top1 = jnp.max(probs, axis=-1, keepdims=True)
    e1 = jnp.min(jnp.where(probs >= top1, idx, E), axis=-1, keepdims=True)
    m1 = idx == e1
    probs2 = jnp.where(m1, -jnp.inf, probs)
    top2 = jnp.max(probs2, axis=-1, keepdims=True)
    e2 = jnp.min(jnp.where(probs2 >= top2, idx, E), axis=-1, keepdims=True)
    m2 = idx == e2
    denom = top1 + top2
    coeff = (jnp.where(m1, top1, 0.0) + jnp.where(m2, top2, 0.0)) / denom
    ep = pb_ref[...].reshape(E, B, EXP_D) * inv_s
    acc = jnp.zeros((B, C), jnp.float32)
    for e in range(E):
        y = ep[e] @ expw_ref[e] + expb_ref[e]
        acc = acc + coeff[:, e:e + 1] * y
    out_ref[...] = acc


def _tc_combine(pa, pb, gate_W1, gate_b1, gate_W2, gate_b2, exp_W, exp_b):
    return pl.pallas_call(
        _tc_body,
        out_shape=jax.ShapeDtypeStruct((B, C), jnp.float32),
    )(pa, pb, gate_W1, gate_b1, gate_W2, gate_b2, exp_W, exp_b)


def kernel(x, emb, gate_W1, gate_b1, gate_W2, gate_b2, exp_emb, exp_W, exp_b):
    vocab = emb.shape[0]
    x_flat = x.reshape(-1).astype(jnp.int32)
    eemb_flat = exp_emb.reshape(E * vocab, EXP_D)
    pa, pb = _sc_pools(x_flat, emb, eemb_flat)
    return _tc_combine(pa, pb, gate_W1, gate_b1, gate_W2, gate_b2,
                       exp_W, exp_b)
